# baseline (speedup 1.0000x reference)
import functools

import jax
import jax.numpy as jnp
from jax import lax
from jax.experimental import pallas as pl
from jax.experimental.pallas import tpu as pltpu
from jax.experimental.pallas import tpu_sc as plsc

N_NODES = 10000
D_FEAT = 128
N_EDGES = 320000
BN_EPS = 1e-5

_NC = 2
_NS = 16
_NW = _NC * _NS
_K = 64
_NB = 4
_CH = 160
_EW = _K * _CH
_EPAD = _EW * _NW
_NPAD = 10240
_RT = _NPAD // _NS


def _make_sc_agg():
    mesh = plsc.VectorSubcoreMesh(core_axis_name="c", subcore_axis_name="s")

    @functools.partial(
        pl.kernel,
        mesh=mesh,
        out_type=jax.ShapeDtypeStruct((_NC, _NPAD, D_FEAT), jnp.float32),
        scratch_types=(
            [pltpu.VMEM((_EW,), jnp.int32)]
            + [pltpu.VMEM((_K,), jnp.int32) for _ in range(_NB)]
            + [pltpu.VMEM((_K,), jnp.int32) for _ in range(_NB)]
            + [pltpu.VMEM((_K, D_FEAT), jnp.float32) for _ in range(_NB)]
            + [pltpu.VMEM_SHARED((_NPAD, D_FEAT), jnp.float32)]
            + [pltpu.SemaphoreType.DMA for _ in range(2 * _NB)]
        ),
    )
    def sc_agg(x_hbm, combo_hbm, out_hbm, combo_v, *bufs):
        srcs = bufs[0:_NB]
        dsts = bufs[_NB:2 * _NB]
        gbs = bufs[2 * _NB:3 * _NB]
        agg = bufs[3 * _NB]
        semg = bufs[3 * _NB + 1:3 * _NB + 1 + _NB]
        sems = bufs[3 * _NB + 1 + _NB:3 * _NB + 1 + 2 * _NB]
        cid = lax.axis_index("c")
        sid = lax.axis_index("s")
        wid = sid * _NC + cid

        z16 = jnp.zeros((16,), jnp.float32)

        def zbody(i, _):
            r = jnp.int32(i) // (D_FEAT // 16)
            c = jnp.int32(i) % (D_FEAT // 16)
            gbs[0][r, pl.ds(c * 16, 16)] = z16
            return 0

        lax.fori_loop(jnp.int32(0), jnp.int32(_K * D_FEAT // 16), zbody, 0)
        for q in range(_RT // _K):
            pltpu.sync_copy(gbs[0], agg.at[pl.ds(sid * _RT + q * _K, _K)])

        pltpu.sync_copy(combo_hbm.at[wid], combo_v)
        plsc.subcore_barrier()

        def stage_gather(c, b):
            for v in range(_K // 16):
                cv = combo_v[pl.ds(c * _K + v * 16, 16)]
                srcs[b][pl.ds(v * 16, 16)] = lax.shift_right_logical(
                    cv, jnp.int32(16))
                dsts[b][pl.ds(v * 16, 16)] = lax.bitwise_and(
                    cv, jnp.int32(0xFFFF))
            pltpu.async_copy(x_hbm.at[srcs[b]], gbs[b], semg[b])

        for b in range(_NB):
            stage_gather(jnp.int32(b), b)

        def body(jj, _):
            j = jnp.int32(jj) * _NB
            for b in range(_NB):
                pltpu.make_async_copy(x_hbm.at[srcs[b]], gbs[b],
                                      semg[b]).wait()
                pltpu.async_copy(gbs[b], agg.at[dsts[b]], sems[b], add=True)
            for b in range(_NB):
                c2 = j + b + _NB

                @pl.when(c2 < _CH)
                def _(b=b, c2=c2):
                    pltpu.make_async_copy(gbs[b], agg.at[dsts[b]],
                                          sems[b]).wait()
                    stage_gather(c2, b)

            return 0

        lax.fori_loop(jnp.int32(0), jnp.int32(_CH // _NB), body, 0)
        for b in range(_NB):
            pltpu.make_async_copy(gbs[b], agg.at[dsts[b]], sems[b]).wait()
        plsc.subcore_barrier()

        pltpu.sync_copy(agg.at[pl.ds(sid * _RT, _RT)],
                        out_hbm.at[cid, pl.ds(sid * _RT, _RT)])

    return sc_agg


def _tc_finish(x_ref, agg_ref, w_ref, b_ref, g_ref, bt_ref, out_ref):
    h = x_ref[...] + agg_ref[0, :N_NODES] + agg_ref[1, :N_NODES]
    lin = lax.dot_general(h, w_ref[...], (((1,), (1,)), ((), ())),
                          preferred_element_type=jnp.float32) + b_ref[...]
    mean = jnp.mean(lin, axis=0, keepdims=True)
    cent = lin - mean
    var = jnp.mean(cent * cent, axis=0, keepdims=True)
    out_ref[...] = cent * lax.rsqrt(var + BN_EPS) * g_ref[...] + bt_ref[...]


def kernel(x, edge_index, W, b, gamma, beta):
    ei = edge_index.astype(jnp.int32)
    pad = _EPAD - N_EDGES
    pad_i = jnp.arange(pad, dtype=jnp.int32)
    combo_pad = jnp.bitwise_or(jnp.left_shift(pad_i % N_NODES, 16),
                               N_NODES + pad_i % (_NPAD - N_NODES))
    combo = jnp.bitwise_or(jnp.left_shift(ei[0], 16), ei[1])
    combo3 = jnp.concatenate([combo, combo_pad]).reshape(_NW, _EW)

    agg = _make_sc_agg()(x, combo3)

    out = pl.pallas_call(
        _tc_finish,
        out_shape=jax.ShapeDtypeStruct((N_NODES, D_FEAT), jnp.float32),
    )(x, agg, W, b.reshape(1, D_FEAT), gamma.reshape(1, D_FEAT),
      beta.reshape(1, D_FEAT))
    return out


# speedup vs baseline: 1.0405x; 1.0405x over previous
import functools

import jax
import jax.numpy as jnp
from jax import lax
from jax.experimental import pallas as pl
from jax.experimental.pallas import tpu as pltpu
from jax.experimental.pallas import tpu_sc as plsc

N_NODES = 10000
D_FEAT = 128
N_EDGES = 320000
BN_EPS = 1e-5

_NC = 2
_NS = 16
_NW = _NC * _NS
_K = 128
_CH = 80
_EW = _K * _CH
_EPAD = _EW * _NW
_NPAD = 10240
_RT = _NPAD // _NS


def _make_sc_agg():
    mesh = plsc.VectorSubcoreMesh(core_axis_name="c", subcore_axis_name="s")

    @functools.partial(
        pl.kernel,
        mesh=mesh,
        out_type=jax.ShapeDtypeStruct((_NC, _NPAD, D_FEAT), jnp.float32),
        scratch_types=[
            pltpu.VMEM((_CH, _K), jnp.int32),
            pltpu.VMEM((_K,), jnp.int32),
            pltpu.VMEM((_K,), jnp.int32),
            pltpu.VMEM((_K,), jnp.int32),
            pltpu.VMEM((_K,), jnp.int32),
            pltpu.VMEM((_K, D_FEAT), jnp.float32),
            pltpu.VMEM((_K, D_FEAT), jnp.float32),
            pltpu.VMEM_SHARED((_NPAD, D_FEAT), jnp.float32),
            pltpu.SemaphoreType.DMA,
            pltpu.SemaphoreType.DMA,
        ],
    )
    def sc_agg(x_hbm, combo_hbm, out_hbm,
               combo_v, src0, src1, dst0, dst1, buf0, buf1, agg, sem0, sem1):
        cid = lax.axis_index("c")
        sid = lax.axis_index("s")
        wid = sid * _NC + cid

        z16 = jnp.zeros((16,), jnp.float32)

        def zbody(i, _):
            r = jnp.int32(i) // (D_FEAT // 16)
            c = jnp.int32(i) % (D_FEAT // 16)
            buf0[r, pl.ds(c * 16, 16)] = z16
            return 0

        lax.fori_loop(jnp.int32(0), jnp.int32(_K * D_FEAT // 16), zbody, 0)
        for q in range(_RT // _K):
            pltpu.sync_copy(buf0, agg.at[pl.ds(sid * _RT + q * _K, _K)])

        pltpu.sync_copy(combo_hbm.at[wid], combo_v)
        plsc.subcore_barrier()

        def unpack(j, src_c, dst_c):
            for v in range(_K // 16):
                cv = combo_v[j, pl.ds(v * 16, 16)]
                src_c[pl.ds(v * 16, 16)] = lax.shift_right_logical(
                    cv, jnp.int32(16))
                dst_c[pl.ds(v * 16, 16)] = lax.bitwise_and(cv, jnp.int32(0xFFFF))

        unpack(jnp.int32(0), src0, dst0)
        pltpu.async_copy(x_hbm.at[src0], buf0, sem0)

        def body(jj, _):
            j = jnp.int32(jj) * 2
            unpack(j + 1, src1, dst1)
            pltpu.async_copy(x_hbm.at[src1], buf1, sem1)
            pltpu.make_async_copy(x_hbm.at[src0], buf0, sem0).wait()
            pltpu.sync_copy(buf0, agg.at[dst0], add=True)

            @pl.when(j + 2 < _CH)
            def _():
                unpack(j + 2, src0, dst0)
                pltpu.async_copy(x_hbm.at[src0], buf0, sem0)

            pltpu.make_async_copy(x_hbm.at[src1], buf1, sem1).wait()
            pltpu.sync_copy(buf1, agg.at[dst1], add=True)
            return 0

        lax.fori_loop(jnp.int32(0), jnp.int32(_CH // 2), body, 0)
        plsc.subcore_barrier()

        pltpu.sync_copy(agg.at[pl.ds(sid * _RT, _RT)],
                        out_hbm.at[cid, pl.ds(sid * _RT, _RT)])

    return sc_agg


def _tc_finish(x_ref, agg_ref, w_ref, b_ref, g_ref, bt_ref, out_ref):
    h = x_ref[...] + agg_ref[0, :N_NODES] + agg_ref[1, :N_NODES]
    lin = lax.dot_general(h, w_ref[...], (((1,), (1,)), ((), ())),
                          preferred_element_type=jnp.float32) + b_ref[...]
    mean = jnp.mean(lin, axis=0, keepdims=True)
    cent = lin - mean
    var = jnp.mean(cent * cent, axis=0, keepdims=True)
    out_ref[...] = cent * lax.rsqrt(var + BN_EPS) * g_ref[...] + bt_ref[...]


def kernel(x, edge_index, W, b, gamma, beta):
    ei = edge_index.astype(jnp.int32)
    pad = _EPAD - N_EDGES
    pad_i = jnp.arange(pad, dtype=jnp.int32)
    combo_pad = jnp.bitwise_or(jnp.left_shift(pad_i % N_NODES, 16),
                               N_NODES + pad_i % (_NPAD - N_NODES))
    combo = jnp.bitwise_or(jnp.left_shift(ei[0], 16), ei[1])
    combo3 = jnp.concatenate([combo, combo_pad]).reshape(_NW, _CH, _K)

    agg = _make_sc_agg()(x, combo3)

    out = pl.pallas_call(
        _tc_finish,
        out_shape=jax.ShapeDtypeStruct((N_NODES, D_FEAT), jnp.float32),
    )(x, agg, W, b.reshape(1, D_FEAT), gamma.reshape(1, D_FEAT),
      beta.reshape(1, D_FEAT))
    return out
